# baseline (speedup 1.0000x reference)
import functools

import jax
import jax.numpy as jnp
from jax import lax
from jax.experimental import pallas as pl
from jax.experimental.pallas import tpu as pltpu
from jax.experimental.pallas import tpu_sc as plsc

D_MODEL = 64
D_PAD = 128
SCALE = 8.0

_NC, _NS = 2, 16
_NW = _NC * _NS
_B = 4096 * 200
_IDX_W = 128
_KSTAGE = 8
_STAGE = _KSTAGE * _IDX_W
_K = 4
_CHUNK = _K * _IDX_W
_PER_W = _B // _NW
_NSTAGE = _PER_W // _STAGE


def _emb_body(x_hbm, table_hbm, out_hbm, idx_v, rows_v, gsem):
    wid = lax.axis_index("s") * _NC + lax.axis_index("c")
    row_base = pl.multiple_of(wid * _PER_W, _STAGE)
    blk_base = row_base // _IDX_W

    @pl.loop(0, _NSTAGE)
    def _stage(i):
        blk_off = pl.multiple_of(blk_base + i * _KSTAGE, _KSTAGE)
        pltpu.sync_copy(x_hbm.at[pl.ds(blk_off, _KSTAGE)], idx_v)

        for h in range(2):
            copies = []
            for j in range(_K):
                copies.append(
                    pltpu.async_copy(
                        table_hbm.at[idx_v.at[h * _K + j]],
                        rows_v.at[pl.ds(j * _IDX_W, _IDX_W)],
                        gsem,
                    )
                )
            for c in copies:
                c.wait()

            @pl.loop(0, _CHUNK // 2, unroll=2)
            def _pack(q):
                for g in range(D_MODEL // 16):
                    lo = pl.ds(g * 16, 16)
                    hi = pl.ds(D_MODEL + g * 16, 16)
                    a = rows_v[2 * q, lo] * SCALE
                    b = rows_v[2 * q + 1, lo] * SCALE
                    rows_v[q, lo] = a
                    rows_v[q, hi] = b

            pck_off = pl.multiple_of(
                (row_base + i * _STAGE + h * _CHUNK) // 2, _CHUNK // 2
            )
            pltpu.sync_copy(
                rows_v.at[pl.ds(0, _CHUNK // 2)],
                out_hbm.at[pl.ds(pck_off, _CHUNK // 2)],
            )


@jax.jit
def _emb(xf, table_pad):
    mesh = plsc.VectorSubcoreMesh(
        core_axis_name="c", subcore_axis_name="s",
        num_cores=_NC, num_subcores=_NS,
    )
    f = pl.kernel(
        _emb_body,
        out_type=jax.ShapeDtypeStruct((_B // 2, D_PAD), jnp.float32),
        mesh=mesh,
        scratch_types=[
            pltpu.VMEM((_KSTAGE, _IDX_W), jnp.int32),
            pltpu.VMEM((_CHUNK, D_PAD), jnp.float32),
            pltpu.SemaphoreType.DMA,
        ],
    )
    return f(xf, table_pad)


def kernel(x, table):
    xf = x.reshape(_B // _IDX_W, _IDX_W)
    table_pad = jnp.pad(table, ((0, 0), (0, D_PAD - D_MODEL)))
    out = _emb(xf, table_pad)
    return out.reshape(x.shape + (D_MODEL,))
